# baseline (speedup 1.0000x reference)
import functools

import jax
import jax.numpy as jnp
from jax import lax
from jax.experimental import pallas as pl
from jax.experimental.pallas import tpu as pltpu
from jax.experimental.pallas import tpu_sc as plsc

_EMB = 64
_NC = 2
_NS = 16
_NW = _NC * _NS

_CH = 512
_SUB = 128
_NSUB = _CH // _SUB


def _gather_body(idx_hbm, table_hbm, out_hbm, idx_v, rows_v, sem):
  num_rows = out_hbm.shape[0]
  per_w = num_rows // _NW
  n_steps = per_w // _CH
  wid = lax.axis_index("s") * _NC + lax.axis_index("c")
  base = wid * per_w

  def step(i, carry):
    off = base + i * _CH
    pltpu.sync_copy(idx_hbm.at[pl.ds(off, _CH)], idx_v)
    copies = [
        pltpu.async_copy(
            table_hbm.at[idx_v.at[pl.ds(j * _SUB, _SUB)]],
            rows_v.at[pl.ds(j * _SUB, _SUB)],
            sem,
        )
        for j in range(_NSUB)
    ]
    for c in copies:
      c.wait()
    pltpu.sync_copy(rows_v, out_hbm.at[pl.ds(off, _CH)])
    return carry

  lax.fori_loop(0, n_steps, step, 0)


@functools.partial(jax.jit, static_argnums=(2,))
def _embed_lookup(idx_flat, table, num_rows):
  run = pl.kernel(
      _gather_body,
      out_type=jax.ShapeDtypeStruct((num_rows, _EMB), jnp.float32),
      mesh=plsc.VectorSubcoreMesh(
          core_axis_name="c", subcore_axis_name="s",
          num_cores=_NC, num_subcores=_NS,
      ),
      scratch_types=[
          pltpu.VMEM((_CH,), jnp.int32),
          pltpu.VMEM((_CH, _EMB), jnp.float32),
          pltpu.SemaphoreType.DMA,
      ],
      compiler_params=pltpu.CompilerParams(use_tc_tiling_on_sc=False),
  )
  return run(idx_flat, table)


def kernel(input_tensor, token_table):
  b, s = input_tensor.shape
  idx_flat = input_tensor.reshape(-1).astype(jnp.int32)
  out = _embed_lookup(idx_flat, token_table, b * s)
  return out.reshape(b, s, _EMB)


# speedup vs baseline: 1.0741x; 1.0741x over previous
import functools

import jax
import jax.numpy as jnp
from jax import lax
from jax.experimental import pallas as pl
from jax.experimental.pallas import tpu as pltpu
from jax.experimental.pallas import tpu_sc as plsc

_EMB = 64
_NC = 2
_NS = 16
_NW = _NC * _NS

_CH = 512
_SUB = 128
_NSUB = _CH // _SUB


def _gather_body(idx_hbm, table_hbm, out_hbm,
                 idx0, idx1, rows0, rows1,
                 si0, si1, sg0, sg1, so0, so1):
  num_rows = out_hbm.shape[0]
  per_w = num_rows // _NW
  n_steps = per_w // _CH
  wid = lax.axis_index("s") * _NC + lax.axis_index("c")
  base = wid * per_w

  idxs = (idx0, idx1)
  rows = (rows0, rows1)
  sem_i = (si0, si1)
  sem_g = (sg0, sg1)
  sem_o = (so0, so1)

  def fire_gathers(s):
    for j in range(_NSUB):
      pltpu.async_copy(
          table_hbm.at[idxs[s].at[pl.ds(j * _SUB, _SUB)]],
          rows[s].at[pl.ds(j * _SUB, _SUB)],
          sem_g[s],
      )

  def wait_gathers(s):
    pltpu.make_async_copy(
        table_hbm.at[pl.ds(0, _CH)], rows[s], sem_g[s]).wait()

  def wait_idx(s):
    pltpu.make_async_copy(
        idx_hbm.at[pl.ds(base, _CH)], idxs[s], sem_i[s]).wait()

  def wait_store(s):
    pltpu.make_async_copy(
        rows[s], out_hbm.at[pl.ds(base, _CH)], sem_o[s]).wait()

  def step(i, s):
    off = base + i * _CH
    wait_idx(s)

    @pl.when(i >= 2)
    def _():
      wait_store(s)

    fire_gathers(s)

    @pl.when(i >= 1)
    def _():
      wait_gathers(s ^ 1)

    @pl.when(i + 1 < n_steps)
    def _():
      pltpu.async_copy(
          idx_hbm.at[pl.ds(off + _CH, _CH)], idxs[s ^ 1], sem_i[s ^ 1])

    @pl.when(i >= 1)
    def _():
      pltpu.async_copy(
          rows[s ^ 1], out_hbm.at[pl.ds(off - _CH, _CH)], sem_o[s ^ 1])

  pltpu.async_copy(idx_hbm.at[pl.ds(base, _CH)], idxs[0], sem_i[0])

  def pair(k, carry):
    step(2 * k, 0)
    step(2 * k + 1, 1)
    return carry

  lax.fori_loop(0, n_steps // 2, pair, 0)

  last_off = base + (n_steps - 1) * _CH
  wait_gathers(1)
  pltpu.async_copy(rows[1], out_hbm.at[pl.ds(last_off, _CH)], sem_o[1])
  wait_store(0)
  wait_store(1)


@functools.partial(jax.jit, static_argnums=(2,))
def _embed_lookup(idx_flat, table, num_rows):
  run = pl.kernel(
      _gather_body,
      out_type=jax.ShapeDtypeStruct((num_rows, _EMB), jnp.float32),
      mesh=plsc.VectorSubcoreMesh(
          core_axis_name="c", subcore_axis_name="s",
          num_cores=_NC, num_subcores=_NS,
      ),
      scratch_types=[
          pltpu.VMEM((_CH,), jnp.int32),
          pltpu.VMEM((_CH,), jnp.int32),
          pltpu.VMEM((_CH, _EMB), jnp.float32),
          pltpu.VMEM((_CH, _EMB), jnp.float32),
          pltpu.SemaphoreType.DMA,
          pltpu.SemaphoreType.DMA,
          pltpu.SemaphoreType.DMA,
          pltpu.SemaphoreType.DMA,
          pltpu.SemaphoreType.DMA,
          pltpu.SemaphoreType.DMA,
      ],
      compiler_params=pltpu.CompilerParams(use_tc_tiling_on_sc=False),
  )
  return run(idx_flat, table)


def kernel(input_tensor, token_table):
  b, s = input_tensor.shape
  idx_flat = input_tensor.reshape(-1).astype(jnp.int32)
  out = _embed_lookup(idx_flat, token_table, b * s)
  return out.reshape(b, s, _EMB)
